# speedup vs baseline: 6.8615x; 6.8615x over previous
import functools

import jax
import jax.numpy as jnp
from jax import lax
from jax.experimental import pallas as pl

B = 512
D_FEAT = 128
D_EDGE = 16
NODE_BLK = 2000
EDGE_BLK = 4000


def _node_body(batch_ref, x_ref, nsum_ref, ncnt_ref, nlt_ref):
    i = pl.program_id(0)

    @pl.when(i == 0)
    def _init():
        nsum_ref[...] = jnp.zeros_like(nsum_ref)
        ncnt_ref[...] = jnp.zeros_like(ncnt_ref)
        nlt_ref[...] = jnp.zeros_like(nlt_ref)

    b_ids = batch_ref[0, :, :]
    seg = lax.broadcasted_iota(jnp.int32, (1, B), 1)
    eq = (b_ids == seg).astype(jnp.float32)
    lt = (b_ids < seg).astype(jnp.float32)

    dn = (((0,), (0,)), ((), ()))
    nsum_ref[...] += lax.dot_general(eq, x_ref[...], dn,
                                     preferred_element_type=jnp.float32)
    ones = jnp.ones((NODE_BLK, 1), jnp.float32)
    ncnt_ref[...] += lax.dot_general(eq, ones, dn,
                                     preferred_element_type=jnp.float32)
    nlt_ref[...] += jnp.sum(lt, axis=0, keepdims=True)


def _edge_body(src_ref, attr_ref, nlt_ref, esum_ref, ecnt_ref):
    i = pl.program_id(0)

    @pl.when(i == 0)
    def _init():
        esum_ref[...] = jnp.zeros_like(esum_ref)
        ecnt_ref[...] = jnp.zeros_like(ecnt_ref)

    srcf = src_ref[0, :, :].astype(jnp.float32)
    starts = nlt_ref[...]
    ge = (srcf >= starts).astype(jnp.float32)

    dn = (((0,), (0,)), ((), ()))
    esum_ref[...] += lax.dot_general(ge, attr_ref[...], dn,
                                     preferred_element_type=jnp.float32)
    ones = jnp.ones((EDGE_BLK, 1), jnp.float32)
    ecnt_ref[...] += lax.dot_general(ge, ones, dn,
                                     preferred_element_type=jnp.float32)


def _final_body(u_ref, nsum_ref, ncnt_ref, esum_ref, ecnt_ref,
                W1_ref, b1_ref, W2_ref, b2_ref, W3_ref, b3_ref, out_ref):
    nmean = nsum_ref[...] / jnp.maximum(ncnt_ref[...], 1.0)

    esum_ge = esum_ref[...]
    ecnt_ge = ecnt_ref[...]
    zrow16 = jnp.zeros((1, D_EDGE), jnp.float32)
    zrow1 = jnp.zeros((1, 1), jnp.float32)
    esum = esum_ge - jnp.concatenate([esum_ge[1:], zrow16], axis=0)
    ecnt = ecnt_ge - jnp.concatenate([ecnt_ge[1:], zrow1], axis=0)
    emean = esum / jnp.maximum(ecnt, 1.0)

    W1 = W1_ref[...]
    h = (u_ref[...] @ W1[0:128]
         + nmean @ W1[128:256]
         + emean @ W1[256:272]
         + b1_ref[...])
    h = jnp.maximum(h, 0.0)
    h = jnp.maximum(h @ W2_ref[...] + b2_ref[...], 0.0)
    out_ref[...] = h @ W3_ref[...] + b3_ref[...]


@jax.jit
def kernel(x, edge_index, edge_attr, u, batch, W1, b1, W2, b2, W3, b3):
    n_nodes = x.shape[0]
    n_edges = edge_attr.shape[0]
    n_nb = n_nodes // NODE_BLK
    n_eb = n_edges // EDGE_BLK

    batch3 = batch.astype(jnp.int32).reshape(n_nb, NODE_BLK, 1)
    src3 = edge_index[0].astype(jnp.int32).reshape(n_eb, EDGE_BLK, 1)

    nsum, ncnt, nlt = pl.pallas_call(
        _node_body,
        grid=(n_nb,),
        in_specs=[
            pl.BlockSpec((1, NODE_BLK, 1), lambda i: (i, 0, 0)),
            pl.BlockSpec((NODE_BLK, D_FEAT), lambda i: (i, 0)),
        ],
        out_specs=[
            pl.BlockSpec((B, D_FEAT), lambda i: (0, 0)),
            pl.BlockSpec((B, 1), lambda i: (0, 0)),
            pl.BlockSpec((1, B), lambda i: (0, 0)),
        ],
        out_shape=[
            jax.ShapeDtypeStruct((B, D_FEAT), jnp.float32),
            jax.ShapeDtypeStruct((B, 1), jnp.float32),
            jax.ShapeDtypeStruct((1, B), jnp.float32),
        ],
    )(batch3, x)

    esum_ge, ecnt_ge = pl.pallas_call(
        _edge_body,
        grid=(n_eb,),
        in_specs=[
            pl.BlockSpec((1, EDGE_BLK, 1), lambda i: (i, 0, 0)),
            pl.BlockSpec((EDGE_BLK, D_EDGE), lambda i: (i, 0)),
            pl.BlockSpec((1, B), lambda i: (0, 0)),
        ],
        out_specs=[
            pl.BlockSpec((B, D_EDGE), lambda i: (0, 0)),
            pl.BlockSpec((B, 1), lambda i: (0, 0)),
        ],
        out_shape=[
            jax.ShapeDtypeStruct((B, D_EDGE), jnp.float32),
            jax.ShapeDtypeStruct((B, 1), jnp.float32),
        ],
    )(src3, edge_attr, nlt)

    out = pl.pallas_call(
        _final_body,
        out_shape=jax.ShapeDtypeStruct((B, D_FEAT), jnp.float32),
    )(u, nsum, ncnt, esum_ge, ecnt_ge, W1, b1, W2, b2, W3, b3)
    return out


# speedup vs baseline: 13.3268x; 1.9422x over previous
import functools

import jax
import jax.numpy as jnp
from jax import lax
from jax.experimental import pallas as pl
from jax.experimental.pallas import tpu as pltpu
from jax.experimental.pallas import tpu_sc as plsc

B = 512
D_FEAT = 128
D_EDGE = 16
NODE_BLK = 2000

N_WORKERS = 32
EDGE_CHUNK = 2000
ACC = B * D_EDGE


def _node_body(batch_ref, x_ref, nsum_ref, ncnt_ref):
    i = pl.program_id(0)

    @pl.when(i == 0)
    def _init():
        nsum_ref[...] = jnp.zeros_like(nsum_ref)
        ncnt_ref[...] = jnp.zeros_like(ncnt_ref)

    b_ids = batch_ref[0, :, :]
    seg = lax.broadcasted_iota(jnp.int32, (1, B), 1)
    eq = (b_ids == seg).astype(jnp.float32)

    dn = (((0,), (0,)), ((), ()))
    nsum_ref[...] += lax.dot_general(eq, x_ref[...], dn,
                                     preferred_element_type=jnp.float32)
    ones = jnp.ones((NODE_BLK, 1), jnp.float32)
    ncnt_ref[...] += lax.dot_general(eq, ones, dn,
                                     preferred_element_type=jnp.float32)


def _make_sc_edge(n_edges):
    edges_per_w = n_edges // N_WORKERS
    n_chunks = edges_per_w // EDGE_CHUNK
    groups = EDGE_CHUNK // 16

    mesh = plsc.VectorSubcoreMesh(core_axis_name="c", subcore_axis_name="s")

    @functools.partial(
        pl.kernel,
        out_type=(jax.ShapeDtypeStruct((N_WORKERS, ACC), jnp.float32),
                  jax.ShapeDtypeStruct((N_WORKERS, ACC), jnp.float32)),
        mesh=mesh,
        scratch_types=[
            pltpu.VMEM((50000,), jnp.int32),
            pltpu.VMEM((EDGE_CHUNK,), jnp.int32),
            pltpu.VMEM((EDGE_CHUNK * D_EDGE,), jnp.float32),
            pltpu.VMEM((ACC,), jnp.float32),
            pltpu.VMEM((ACC,), jnp.float32),
        ],
        compiler_params=pltpu.CompilerParams(needs_layout_passes=False),
    )
    def sc_edge(src_hbm, bp_hbm, attr_hbm, acc_hbm, cnt_hbm,
                bp_v, src_v, attr_v, acc_v, cnt_v):
        wid = lax.axis_index("s") * 2 + lax.axis_index("c")
        pltpu.sync_copy(bp_hbm, bp_v)

        z16 = jnp.zeros((16,), jnp.float32)

        def zbody(i, carry):
            acc_v[pl.ds(i * 16, 16)] = z16
            cnt_v[pl.ds(i * 16, 16)] = z16
            return carry

        lax.fori_loop(0, ACC // 16, zbody, 0)

        iota = lax.iota(jnp.int32, 16)
        lane_base = iota * B
        riota = iota * D_EDGE
        ones = jnp.ones((16,), jnp.float32)
        wbase = wid * edges_per_w

        def chunk_body(c, carry):
            base = wbase + c * EDGE_CHUNK
            pltpu.sync_copy(src_hbm.at[pl.ds(base, EDGE_CHUNK)], src_v)
            pltpu.sync_copy(
                attr_hbm.at[pl.ds(base * D_EDGE, EDGE_CHUNK * D_EDGE)],
                attr_v)

            def gbody(g, gcarry):
                s = src_v[pl.ds(g * 16, 16)]
                word = plsc.load_gather(bp_v, [s >> 1])
                shift = (s & 1) << 4
                seg = (word >> shift) & 0xFFFF
                base16 = seg * D_EDGE
                rowidx = g * (16 * D_EDGE) + riota
                for d in range(D_EDGE):
                    vals = plsc.load_gather(attr_v, [rowidx + d])
                    plsc.addupdate_scatter(acc_v, [base16 + d], vals)
                plsc.addupdate_scatter(cnt_v, [lane_base + seg], ones)
                return gcarry

            lax.fori_loop(0, groups, gbody, carry)
            return carry

        lax.fori_loop(0, n_chunks, chunk_body, 0)
        pltpu.sync_copy(acc_v, acc_hbm.at[wid])
        pltpu.sync_copy(cnt_v, cnt_hbm.at[wid])

    return sc_edge


def _final_body(u_ref, nsum_ref, ncnt_ref, eacc_ref, ecnt_ref,
                W1_ref, b1_ref, W2_ref, b2_ref, W3_ref, b3_ref, out_ref):
    nmean = nsum_ref[...] / jnp.maximum(ncnt_ref[...], 1.0)

    esum = jnp.sum(eacc_ref[...], axis=0)
    cnt16 = jnp.sum(ecnt_ref[...], axis=0)
    dn = (((0,), (0,)), ((), ()))
    ecnt_col = lax.dot_general(cnt16, jnp.ones((16, 1), jnp.float32), dn,
                               preferred_element_type=jnp.float32)
    emean = esum / jnp.maximum(ecnt_col, 1.0)

    W1 = W1_ref[...]
    h = (u_ref[...] @ W1[0:128]
         + nmean @ W1[128:256]
         + emean @ W1[256:272]
         + b1_ref[...])
    h = jnp.maximum(h, 0.0)
    h = jnp.maximum(h @ W2_ref[...] + b2_ref[...], 0.0)
    out_ref[...] = h @ W3_ref[...] + b3_ref[...]


@jax.jit
def kernel(x, edge_index, edge_attr, u, batch, W1, b1, W2, b2, W3, b3):
    n_nodes = x.shape[0]
    n_edges = edge_attr.shape[0]
    n_nb = n_nodes // NODE_BLK

    b32 = batch.astype(jnp.int32)
    batch3 = b32.reshape(n_nb, NODE_BLK, 1)
    bp = b32[0::2] | (b32[1::2] << 16)
    src = edge_index[0].astype(jnp.int32)

    nsum, ncnt = pl.pallas_call(
        _node_body,
        grid=(n_nb,),
        in_specs=[
            pl.BlockSpec((1, NODE_BLK, 1), lambda i: (i, 0, 0)),
            pl.BlockSpec((NODE_BLK, D_FEAT), lambda i: (i, 0)),
        ],
        out_specs=[
            pl.BlockSpec((B, D_FEAT), lambda i: (0, 0)),
            pl.BlockSpec((B, 1), lambda i: (0, 0)),
        ],
        out_shape=[
            jax.ShapeDtypeStruct((B, D_FEAT), jnp.float32),
            jax.ShapeDtypeStruct((B, 1), jnp.float32),
        ],
    )(batch3, x)

    acc, cnt = _make_sc_edge(n_edges)(src, bp, edge_attr.reshape(-1))
    eacc = acc.reshape(N_WORKERS, B, D_EDGE)
    ecnt = cnt.reshape(N_WORKERS, 16, B)

    out = pl.pallas_call(
        _final_body,
        out_shape=jax.ShapeDtypeStruct((B, D_FEAT), jnp.float32),
    )(u, nsum, ncnt, eacc, ecnt, W1, b1, W2, b2, W3, b3)
    return out


# speedup vs baseline: 16.3703x; 1.2284x over previous
import functools

import jax
import jax.numpy as jnp
from jax import lax
from jax.experimental import pallas as pl
from jax.experimental.pallas import tpu as pltpu
from jax.experimental.pallas import tpu_sc as plsc

B = 512
D_FEAT = 128
D_EDGE = 16
NODE_BLK = 2000

N_WORKERS = 32
EDGE_CHUNK = 2000
ACC = B * D_EDGE


def _node_body(batch_ref, x_ref, nsum_ref, ncnt_ref):
    i = pl.program_id(0)

    @pl.when(i == 0)
    def _init():
        nsum_ref[...] = jnp.zeros_like(nsum_ref)
        ncnt_ref[...] = jnp.zeros_like(ncnt_ref)

    b_ids = batch_ref[0, :, :]
    seg = lax.broadcasted_iota(jnp.int32, (1, B), 1)
    eq = (b_ids == seg).astype(jnp.float32)

    dn = (((0,), (0,)), ((), ()))
    nsum_ref[...] += lax.dot_general(eq, x_ref[...], dn,
                                     preferred_element_type=jnp.float32)
    ones = jnp.ones((NODE_BLK, 1), jnp.float32)
    ncnt_ref[...] += lax.dot_general(eq, ones, dn,
                                     preferred_element_type=jnp.float32)


def _make_sc_edge(n_edges):
    edges_per_w = n_edges // N_WORKERS
    n_chunks = edges_per_w // EDGE_CHUNK
    groups = EDGE_CHUNK // 16

    mesh = plsc.VectorSubcoreMesh(core_axis_name="c", subcore_axis_name="s")

    @functools.partial(
        pl.kernel,
        out_type=(jax.ShapeDtypeStruct((N_WORKERS, ACC), jnp.float32),
                  jax.ShapeDtypeStruct((N_WORKERS, ACC), jnp.float32)),
        mesh=mesh,
        scratch_types=[
            pltpu.VMEM((50000,), jnp.int32),
            pltpu.VMEM((EDGE_CHUNK,), jnp.int32),
            pltpu.VMEM((EDGE_CHUNK, D_EDGE), jnp.float32),
            pltpu.VMEM((ACC,), jnp.float32),
            pltpu.VMEM((ACC,), jnp.float32),
        ],
        compiler_params=pltpu.CompilerParams(needs_layout_passes=False,
                                             use_tc_tiling_on_sc=False),
    )
    def sc_edge(src_hbm, bp_hbm, attr_hbm, acc_hbm, cnt_hbm,
                bp_v, src_v, attr_v, acc_v, cnt_v):
        wid = lax.axis_index("s") * 2 + lax.axis_index("c")
        pltpu.sync_copy(bp_hbm, bp_v)

        z16 = jnp.zeros((16,), jnp.float32)

        def zbody(i, carry):
            acc_v[pl.ds(i * 16, 16)] = z16
            cnt_v[pl.ds(i * 16, 16)] = z16
            return carry

        lax.fori_loop(0, ACC // 16, zbody, 0)

        iota = lax.iota(jnp.int32, 16)
        lane_base = iota * B
        ones = jnp.ones((16,), jnp.float32)
        wbase = wid * edges_per_w

        def chunk_body(c, carry):
            base = wbase + c * EDGE_CHUNK
            pltpu.sync_copy(src_hbm.at[pl.ds(base, EDGE_CHUNK)], src_v)
            pltpu.sync_copy(attr_hbm.at[pl.ds(base, EDGE_CHUNK), :], attr_v)

            @plsc.parallel_loop(0, groups, 1, unroll=2)
            def gbody(g):
                s = src_v[pl.ds(g * 16, 16)]
                word = plsc.load_gather(bp_v, [s >> 1])
                shift = (s & 1) << 4
                seg = (word >> shift) & 0xFFFF
                base16 = seg * D_EDGE
                rows = g * 16 + iota
                for d in range(D_EDGE):
                    vals = plsc.load_gather(
                        attr_v, [rows, jnp.full((16,), d, jnp.int32)])
                    plsc.addupdate_scatter(acc_v, [base16 + d], vals)
                plsc.addupdate_scatter(cnt_v, [lane_base + seg], ones)

            return carry

        lax.fori_loop(0, n_chunks, chunk_body, 0)
        pltpu.sync_copy(acc_v, acc_hbm.at[wid])
        pltpu.sync_copy(cnt_v, cnt_hbm.at[wid])

    return sc_edge


def _final_body(u_ref, nsum_ref, ncnt_ref, eacc_ref, ecnt_ref,
                W1_ref, b1_ref, W2_ref, b2_ref, W3_ref, b3_ref, out_ref):
    nmean = nsum_ref[...] / jnp.maximum(ncnt_ref[...], 1.0)

    esum = jnp.sum(eacc_ref[...], axis=0)
    cnt16 = jnp.sum(ecnt_ref[...], axis=0)
    dn = (((0,), (0,)), ((), ()))
    ecnt_col = lax.dot_general(cnt16, jnp.ones((16, 1), jnp.float32), dn,
                               preferred_element_type=jnp.float32)
    emean = esum / jnp.maximum(ecnt_col, 1.0)

    W1 = W1_ref[...]
    h = (u_ref[...] @ W1[0:128]
         + nmean @ W1[128:256]
         + emean @ W1[256:272]
         + b1_ref[...])
    h = jnp.maximum(h, 0.0)
    h = jnp.maximum(h @ W2_ref[...] + b2_ref[...], 0.0)
    out_ref[...] = h @ W3_ref[...] + b3_ref[...]


@jax.jit
def kernel(x, edge_index, edge_attr, u, batch, W1, b1, W2, b2, W3, b3):
    n_nodes = x.shape[0]
    n_edges = edge_attr.shape[0]
    n_nb = n_nodes // NODE_BLK

    b32 = batch.astype(jnp.int32)
    batch3 = b32.reshape(n_nb, NODE_BLK, 1)
    bp = b32[0::2] | (b32[1::2] << 16)
    src = edge_index[0].astype(jnp.int32)

    nsum, ncnt = pl.pallas_call(
        _node_body,
        grid=(n_nb,),
        in_specs=[
            pl.BlockSpec((1, NODE_BLK, 1), lambda i: (i, 0, 0)),
            pl.BlockSpec((NODE_BLK, D_FEAT), lambda i: (i, 0)),
        ],
        out_specs=[
            pl.BlockSpec((B, D_FEAT), lambda i: (0, 0)),
            pl.BlockSpec((B, 1), lambda i: (0, 0)),
        ],
        out_shape=[
            jax.ShapeDtypeStruct((B, D_FEAT), jnp.float32),
            jax.ShapeDtypeStruct((B, 1), jnp.float32),
        ],
    )(batch3, x)

    acc, cnt = _make_sc_edge(n_edges)(src, bp, edge_attr)
    eacc = acc.reshape(N_WORKERS, B, D_EDGE)
    ecnt = cnt.reshape(N_WORKERS, 16, B)

    out = pl.pallas_call(
        _final_body,
        out_shape=jax.ShapeDtypeStruct((B, D_FEAT), jnp.float32),
    )(u, nsum, ncnt, eacc, ecnt, W1, b1, W2, b2, W3, b3)
    return out


# speedup vs baseline: 22.7232x; 1.3881x over previous
import functools

import jax
import jax.numpy as jnp
from jax import lax
from jax.experimental import pallas as pl
from jax.experimental.pallas import tpu as pltpu
from jax.experimental.pallas import tpu_sc as plsc

B = 512
D_FEAT = 128
D_EDGE = 16
NODE_BLK = 2000

N_WORKERS = 32
EDGE_CHUNK = 2000
ACC = B * D_EDGE
CSTRIDE = B + 1


def _splat(v, j):
    idx = jnp.full((16, 1), j, jnp.int32)
    dn = lax.GatherDimensionNumbers(
        offset_dims=(), collapsed_slice_dims=(0,), start_index_map=(0,))
    return lax.gather(v, idx, dn, slice_sizes=(1,),
                      mode=lax.GatherScatterMode.PROMISE_IN_BOUNDS)


def _node_body(batch_ref, x_ref, nsum_ref, ncnt_ref):
    i = pl.program_id(0)

    @pl.when(i == 0)
    def _init():
        nsum_ref[...] = jnp.zeros_like(nsum_ref)
        ncnt_ref[...] = jnp.zeros_like(ncnt_ref)

    b_ids = batch_ref[0, :, :]
    seg = lax.broadcasted_iota(jnp.int32, (1, B), 1)
    eq = (b_ids == seg).astype(jnp.float32)

    dn = (((0,), (0,)), ((), ()))
    nsum_ref[...] += lax.dot_general(eq, x_ref[...], dn,
                                     preferred_element_type=jnp.float32)
    ones = jnp.ones((NODE_BLK, 1), jnp.float32)
    ncnt_ref[...] += lax.dot_general(eq, ones, dn,
                                     preferred_element_type=jnp.float32)


def _make_sc_edge(n_edges):
    edges_per_w = n_edges // N_WORKERS
    n_chunks = edges_per_w // EDGE_CHUNK
    groups = EDGE_CHUNK // 16

    mesh = plsc.VectorSubcoreMesh(core_axis_name="c", subcore_axis_name="s")

    @functools.partial(
        pl.kernel,
        out_type=(jax.ShapeDtypeStruct((N_WORKERS, ACC), jnp.float32),
                  jax.ShapeDtypeStruct((N_WORKERS, 16 * CSTRIDE), jnp.float32)),
        mesh=mesh,
        scratch_types=[
            pltpu.VMEM((50000,), jnp.int32),
            pltpu.VMEM((EDGE_CHUNK,), jnp.int32),
            pltpu.VMEM((EDGE_CHUNK, D_EDGE), jnp.float32),
            pltpu.VMEM((ACC,), jnp.float32),
            pltpu.VMEM((16 * CSTRIDE,), jnp.float32),
        ],
        compiler_params=pltpu.CompilerParams(needs_layout_passes=False,
                                             use_tc_tiling_on_sc=False),
    )
    def sc_edge(src_hbm, bp_hbm, attr_hbm, acc_hbm, cnt_hbm,
                bp_v, src_v, attr_v, acc_v, cnt_v):
        wid = lax.axis_index("s") * 2 + lax.axis_index("c")
        pltpu.sync_copy(bp_hbm, bp_v)

        z16 = jnp.zeros((16,), jnp.float32)

        def zbody(i, carry):
            acc_v[pl.ds(i * 16, 16)] = z16
            return carry

        def zbody2(i, carry):
            cnt_v[pl.ds(i * 16, 16)] = z16
            return carry

        lax.fori_loop(0, ACC // 16, zbody, 0)
        lax.fori_loop(0, 16 * CSTRIDE // 16, zbody2, 0)

        iota = lax.iota(jnp.int32, 16)
        lane_base = iota * CSTRIDE
        ones = jnp.ones((16,), jnp.float32)
        wbase = wid * edges_per_w

        def chunk_body(c, carry):
            base = wbase + c * EDGE_CHUNK
            pltpu.sync_copy(src_hbm.at[pl.ds(base, EDGE_CHUNK)], src_v)
            pltpu.sync_copy(attr_hbm.at[pl.ds(base, EDGE_CHUNK), :], attr_v)

            @plsc.parallel_loop(0, groups, 1, unroll=2)
            def gbody(g):
                s = src_v[pl.ds(g * 16, 16)]
                word = plsc.load_gather(bp_v, [s >> 1])
                shift = (s & 1) << 4
                seg = (word >> shift) & 0xFFFF
                seg16 = seg * D_EDGE
                for j in range(16):
                    row = attr_v[g * 16 + j, :]
                    plsc.addupdate_scatter(acc_v, [_splat(seg16, j) + iota],
                                           row)
                plsc.addupdate_scatter(cnt_v, [lane_base + seg], ones)

            return carry

        lax.fori_loop(0, n_chunks, chunk_body, 0)
        pltpu.sync_copy(acc_v, acc_hbm.at[wid])
        pltpu.sync_copy(cnt_v, cnt_hbm.at[wid])

    return sc_edge


def _final_body(u_ref, nsum_ref, ncnt_ref, eacc_ref, ecnt_ref,
                W1_ref, b1_ref, W2_ref, b2_ref, W3_ref, b3_ref, out_ref):
    nmean = nsum_ref[...] / jnp.maximum(ncnt_ref[...], 1.0)

    esum = jnp.sum(eacc_ref[...], axis=0)
    cnt16 = jnp.sum(ecnt_ref[...], axis=0)
    dn = (((0,), (0,)), ((), ()))
    ecnt_col = lax.dot_general(cnt16, jnp.ones((16, 1), jnp.float32), dn,
                               preferred_element_type=jnp.float32)
    emean = esum / jnp.maximum(ecnt_col, 1.0)

    W1 = W1_ref[...]
    h = (u_ref[...] @ W1[0:128]
         + nmean @ W1[128:256]
         + emean @ W1[256:272]
         + b1_ref[...])
    h = jnp.maximum(h, 0.0)
    h = jnp.maximum(h @ W2_ref[...] + b2_ref[...], 0.0)
    out_ref[...] = h @ W3_ref[...] + b3_ref[...]


@jax.jit
def kernel(x, edge_index, edge_attr, u, batch, W1, b1, W2, b2, W3, b3):
    n_nodes = x.shape[0]
    n_edges = edge_attr.shape[0]
    n_nb = n_nodes // NODE_BLK

    b32 = batch.astype(jnp.int32)
    batch3 = b32.reshape(n_nb, NODE_BLK, 1)
    bp = b32[0::2] | (b32[1::2] << 16)
    src = edge_index[0].astype(jnp.int32)

    nsum, ncnt = pl.pallas_call(
        _node_body,
        grid=(n_nb,),
        in_specs=[
            pl.BlockSpec((1, NODE_BLK, 1), lambda i: (i, 0, 0)),
            pl.BlockSpec((NODE_BLK, D_FEAT), lambda i: (i, 0)),
        ],
        out_specs=[
            pl.BlockSpec((B, D_FEAT), lambda i: (0, 0)),
            pl.BlockSpec((B, 1), lambda i: (0, 0)),
        ],
        out_shape=[
            jax.ShapeDtypeStruct((B, D_FEAT), jnp.float32),
            jax.ShapeDtypeStruct((B, 1), jnp.float32),
        ],
    )(batch3, x)

    acc, cnt = _make_sc_edge(n_edges)(src, bp, edge_attr)
    eacc = acc.reshape(N_WORKERS, B, D_EDGE)
    ecnt = cnt.reshape(N_WORKERS, 16, CSTRIDE)[:, :, :B]

    out = pl.pallas_call(
        _final_body,
        out_shape=jax.ShapeDtypeStruct((B, D_FEAT), jnp.float32),
    )(u, nsum, ncnt, eacc, ecnt, W1, b1, W2, b2, W3, b3)
    return out
